# baseline (speedup 1.0000x reference)
import functools

import jax
import jax.numpy as jnp
from jax import lax
from jax.experimental import pallas as pl
from jax.experimental.pallas import tpu as pltpu
from jax.experimental.pallas import tpu_sc as plsc

_NUM_CORES = 2
_NUM_SUBCORES = 16
_LANES = 16


def _seq_chunks(seq_len):
    chunks = []
    off = 0
    while off < seq_len:
        size = min(128, seq_len - off)
        chunks.append((off, size))
        off += size
    return chunks


@functools.partial(jax.jit, static_argnames=("batch", "seq", "dim"))
def _sc_mean_pool(content_flat, table, *, batch, seq, dim):
    n_workers = _NUM_CORES * _NUM_SUBCORES
    bpw = batch // n_workers
    assert bpw % 2 == 0
    npair = bpw // 2
    chunks = _seq_chunks(seq)
    nblk = dim // _LANES
    inv_s = jnp.float32(1.0 / seq)

    mesh = plsc.VectorSubcoreMesh(
        core_axis_name="c", subcore_axis_name="s",
        num_cores=_NUM_CORES, num_subcores=_NUM_SUBCORES)

    @functools.partial(
        pl.kernel,
        mesh=mesh,
        out_type=jax.ShapeDtypeStruct((batch, dim), jnp.float32),
        compiler_params=pltpu.CompilerParams(use_tc_tiling_on_sc=False),
        scratch_types=[
            pltpu.VMEM((bpw * seq,), jnp.int32),
            pltpu.VMEM((seq, dim), jnp.float32),
            pltpu.VMEM((seq, dim), jnp.float32),
            pltpu.VMEM((bpw, dim), jnp.float32),
            pltpu.SemaphoreType.DMA,
            pltpu.SemaphoreType.DMA,
        ],
    )
    def mean_pool(content_hbm, table_hbm, out_hbm,
                  idx_v, rows0, rows1, acc_v, sem0, sem1):
        wid = lax.axis_index("s") * _NUM_CORES + lax.axis_index("c")
        base = pl.multiple_of(wid * bpw, 8)

        pltpu.sync_copy(content_hbm.at[pl.ds(base * seq, bpw * seq)], idx_v)

        def gather(b, buf, sem, start):
            off = pl.multiple_of(b * seq, 8)
            for coff, csz in chunks:
                cpy = pltpu.make_async_copy(
                    table_hbm.at[idx_v.at[pl.ds(off + coff, csz)]],
                    buf.at[pl.ds(coff, csz)],
                    sem)
                if start:
                    cpy.start()
                else:
                    cpy.wait()

        def pool(b, buf):
            def body(i, carry):
                s0 = i * 8
                for j in range(8):
                    s = s0 + j
                    carry = tuple(
                        carry[k] + buf[s, pl.ds(k * _LANES, _LANES)]
                        for k in range(nblk))
                return carry
            init = tuple(jnp.zeros((_LANES,), jnp.float32)
                         for _ in range(nblk))
            tail = seq % 8
            acc = lax.fori_loop(0, seq // 8, body, init)
            for s in range(seq - tail, seq):
                acc = tuple(acc[k] + buf[s, pl.ds(k * _LANES, _LANES)]
                            for k in range(nblk))
            for k in range(nblk):
                acc_v[b, pl.ds(k * _LANES, _LANES)] = acc[k] * inv_s

        gather(0, rows0, sem0, start=True)

        def pipeline(p, _):
            b0 = 2 * p
            gather(b0 + 1, rows1, sem1, start=True)
            gather(b0, rows0, sem0, start=False)
            pool(b0, rows0)
            gather(b0 + 2, rows0, sem0, start=True)
            gather(b0 + 1, rows1, sem1, start=False)
            pool(b0 + 1, rows1)
            return 0

        lax.fori_loop(0, npair - 1, pipeline, 0)

        last = bpw - 2
        gather(last + 1, rows1, sem1, start=True)
        gather(last, rows0, sem0, start=False)
        pool(last, rows0)
        gather(last + 1, rows1, sem1, start=False)
        pool(last + 1, rows1)

        pltpu.sync_copy(acc_v, out_hbm.at[pl.ds(base, bpw)])

    return mean_pool(content_flat, table)


def _tc_linear(x, w_pad, b_pad):
    batch, dim = x.shape

    def mm(x_ref, w_ref, b_ref, o_ref):
        o_ref[...] = jnp.dot(
            x_ref[...], w_ref[...],
            preferred_element_type=jnp.float32) + b_ref[...]

    return pl.pallas_call(
        mm,
        out_shape=jax.ShapeDtypeStruct((batch, 128), jnp.float32),
    )(x, w_pad, b_pad)


def kernel(content, table, W, b):
    batch, seq = content.shape
    _, dim = table.shape
    label = W.shape[1]

    content_flat = content.reshape(-1).astype(jnp.int32)
    means = _sc_mean_pool(content_flat, table, batch=batch, seq=seq, dim=dim)

    w_pad = jnp.zeros((dim, 128), jnp.float32).at[:, :label].set(W)
    b_pad = jnp.zeros((1, 128), jnp.float32).at[0, :label].set(b)
    out = _tc_linear(means, w_pad, b_pad)
    return out[:, :label]


# speedup vs baseline: 1.0043x; 1.0043x over previous
import functools

import jax
import jax.numpy as jnp
from jax import lax
from jax.experimental import pallas as pl
from jax.experimental.pallas import tpu as pltpu
from jax.experimental.pallas import tpu_sc as plsc

_NUM_CORES = 2
_NUM_SUBCORES = 16
_LANES = 16


def _seq_chunks(seq_len):
    chunks = []
    off = 0
    while off < seq_len:
        size = min(128, seq_len - off)
        chunks.append((off, size))
        off += size
    return chunks


@functools.partial(jax.jit, static_argnames=("batch", "seq", "dim"))
def _sc_mean_pool(content, table, *, batch, seq, dim):
    n_workers = _NUM_CORES * _NUM_SUBCORES
    bpw = batch // n_workers
    assert bpw % 2 == 0
    npair = bpw // 2
    chunks = _seq_chunks(seq)
    nblk = dim // _LANES
    inv_s = jnp.float32(1.0 / seq)

    mesh = plsc.VectorSubcoreMesh(
        core_axis_name="c", subcore_axis_name="s",
        num_cores=_NUM_CORES, num_subcores=_NUM_SUBCORES)

    @functools.partial(
        pl.kernel,
        mesh=mesh,
        out_type=jax.ShapeDtypeStruct((batch, dim), jnp.float32),
        compiler_params=pltpu.CompilerParams(use_tc_tiling_on_sc=False),
        scratch_types=[
            pltpu.VMEM((bpw, seq), jnp.int32),
            pltpu.VMEM((seq, dim), jnp.float32),
            pltpu.VMEM((seq, dim), jnp.float32),
            pltpu.VMEM((bpw, dim), jnp.float32),
            pltpu.SemaphoreType.DMA,
            pltpu.SemaphoreType.DMA,
        ],
    )
    def mean_pool(content_hbm, table_hbm, out_hbm,
                  idx_v, rows0, rows1, acc_v, sem0, sem1):
        wid = lax.axis_index("s") * _NUM_CORES + lax.axis_index("c")
        base = pl.multiple_of(wid * bpw, 8)

        pltpu.sync_copy(content_hbm.at[pl.ds(base, bpw)], idx_v)

        def gather(b, buf, sem, start):
            for coff, csz in chunks:
                cpy = pltpu.make_async_copy(
                    table_hbm.at[idx_v.at[b, pl.ds(coff, csz)]],
                    buf.at[pl.ds(coff, csz)],
                    sem)
                if start:
                    cpy.start()
                else:
                    cpy.wait()

        def pool(b, buf):
            def body(i, carry):
                s0 = i * 8
                for j in range(8):
                    s = s0 + j
                    carry = tuple(
                        carry[k] + buf[s, pl.ds(k * _LANES, _LANES)]
                        for k in range(nblk))
                return carry
            init = tuple(jnp.zeros((_LANES,), jnp.float32)
                         for _ in range(nblk))
            tail = seq % 8
            acc = lax.fori_loop(0, seq // 8, body, init)
            for s in range(seq - tail, seq):
                acc = tuple(acc[k] + buf[s, pl.ds(k * _LANES, _LANES)]
                            for k in range(nblk))
            for k in range(nblk):
                acc_v[b, pl.ds(k * _LANES, _LANES)] = acc[k] * inv_s

        gather(0, rows0, sem0, start=True)

        def pipeline(p, _):
            b0 = 2 * p
            gather(b0 + 1, rows1, sem1, start=True)
            gather(b0, rows0, sem0, start=False)
            pool(b0, rows0)
            gather(b0 + 2, rows0, sem0, start=True)
            gather(b0 + 1, rows1, sem1, start=False)
            pool(b0 + 1, rows1)
            return 0

        lax.fori_loop(0, npair - 1, pipeline, 0)

        last = bpw - 2
        gather(last + 1, rows1, sem1, start=True)
        gather(last, rows0, sem0, start=False)
        pool(last, rows0)
        gather(last + 1, rows1, sem1, start=False)
        pool(last + 1, rows1)

        pltpu.sync_copy(acc_v, out_hbm.at[pl.ds(base, bpw)])

    return mean_pool(content, table)


def _tc_linear(x, w_pad, b_pad):
    batch, dim = x.shape

    def mm(x_ref, w_ref, b_ref, o_ref):
        o_ref[...] = jnp.dot(
            x_ref[...], w_ref[...],
            preferred_element_type=jnp.float32) + b_ref[...]

    return pl.pallas_call(
        mm,
        out_shape=jax.ShapeDtypeStruct((batch, 128), jnp.float32),
    )(x, w_pad, b_pad)


def kernel(content, table, W, b):
    batch, seq = content.shape
    _, dim = table.shape
    label = W.shape[1]

    means = _sc_mean_pool(content.astype(jnp.int32), table,
                          batch=batch, seq=seq, dim=dim)

    w_pad = jnp.zeros((dim, 128), jnp.float32).at[:, :label].set(W)
    b_pad = jnp.zeros((1, 128), jnp.float32).at[0, :label].set(b)
    out = _tc_linear(means, w_pad, b_pad)
    return out[:, :label]


# speedup vs baseline: 1.8839x; 1.8758x over previous
import functools

import jax
import jax.numpy as jnp
from jax import lax
from jax.experimental import pallas as pl
from jax.experimental.pallas import tpu as pltpu
from jax.experimental.pallas import tpu_sc as plsc

_NUM_CORES = 2
_NUM_SUBCORES = 16
_LANES = 16


def _seq_chunks(seq_len):
    chunks = []
    off = 0
    while off < seq_len:
        size = min(128, seq_len - off)
        chunks.append((off, size))
        off += size
    return chunks


def _tc_table_times_w(tt, w16):
    dim, vocab = tt.shape
    lanes = w16.shape[1]
    cv = 16384
    grid = (vocab + cv - 1) // cv

    rp = 128 // lanes

    def mm(tt_ref, w_ref, o_ref, scr):
        scr[...] = jax.lax.dot_general(
            tt_ref[...], w_ref[...],
            dimension_numbers=(((0,), (0,)), ((), ())),
            preferred_element_type=jnp.float32)
        parts = [scr[pl.Slice(j, cv // rp, rp), :] for j in range(rp)]
        o_ref[...] = jnp.concatenate(parts, axis=1)

    return pl.pallas_call(
        mm,
        grid=(grid,),
        in_specs=[
            pl.BlockSpec((dim, cv), lambda i: (0, i)),
            pl.BlockSpec((dim, lanes), lambda i: (0, 0)),
        ],
        out_specs=pl.BlockSpec((cv // rp, 128), lambda i: (i, 0)),
        out_shape=jax.ShapeDtypeStruct(
            (vocab * lanes // 128, 128), jnp.float32),
        scratch_shapes=[pltpu.VMEM((cv, lanes), jnp.float32)],
    )(tt, w16)


@functools.partial(jax.jit, static_argnames=("batch", "seq", "vocab"))
def _sc_pool_tw(content, tw2d, b16, *, batch, seq, vocab):
    lanes = _LANES
    n_workers = _NUM_CORES * _NUM_SUBCORES
    bpw = batch // n_workers
    assert bpw % 2 == 0
    npair = bpw // 2
    chunks = _seq_chunks(seq)
    inv_s = jnp.float32(1.0 / seq)

    mesh = plsc.VectorSubcoreMesh(
        core_axis_name="c", subcore_axis_name="s",
        num_cores=_NUM_CORES, num_subcores=_NUM_SUBCORES)

    @functools.partial(
        pl.kernel,
        mesh=mesh,
        out_type=jax.ShapeDtypeStruct((batch, lanes), jnp.float32),
        compiler_params=pltpu.CompilerParams(use_tc_tiling_on_sc=False),
        scratch_types=[
            pltpu.VMEM((bpw, seq), jnp.int32),
            pltpu.VMEM((seq, lanes), jnp.float32),
            pltpu.VMEM((seq, lanes), jnp.float32),
            pltpu.VMEM((bpw, lanes), jnp.float32),
            pltpu.VMEM((lanes,), jnp.float32),
            pltpu.SemaphoreType.DMA,
            pltpu.SemaphoreType.DMA,
        ],
    )
    def pool(content_hbm, tw2, b_hbm, out_hbm,
             idx_v, rows0, rows1, acc_v, b_v, sem0, sem1):
        wid = lax.axis_index("s") * _NUM_CORES + lax.axis_index("c")
        base = pl.multiple_of(wid * bpw, 8)

        pltpu.sync_copy(content_hbm.at[pl.ds(base, bpw)], idx_v)
        pltpu.sync_copy(b_hbm, b_v)
        bias = b_v[...]

        def gather(b, buf, sem, start):
            for coff, csz in chunks:
                cpy = pltpu.make_async_copy(
                    tw2.at[idx_v.at[b, pl.ds(coff, csz)]],
                    buf.at[pl.ds(coff, csz)],
                    sem)
                if start:
                    cpy.start()
                else:
                    cpy.wait()

        def pool_one(b, buf):
            def body(i, acc):
                s0 = i * 8
                for j in range(8):
                    acc = acc + buf[s0 + j, :]
                return acc
            acc = lax.fori_loop(0, seq // 8, body,
                                jnp.zeros((lanes,), jnp.float32))
            for s in range(seq - seq % 8, seq):
                acc = acc + buf[s, :]
            acc_v[b, :] = acc * inv_s + bias

        gather(0, rows0, sem0, start=True)

        def pipeline(p, _):
            b0 = 2 * p
            gather(b0 + 1, rows1, sem1, start=True)
            gather(b0, rows0, sem0, start=False)
            pool_one(b0, rows0)
            gather(b0 + 2, rows0, sem0, start=True)
            gather(b0 + 1, rows1, sem1, start=False)
            pool_one(b0 + 1, rows1)
            return 0

        lax.fori_loop(0, npair - 1, pipeline, 0)

        last = bpw - 2
        gather(last + 1, rows1, sem1, start=True)
        gather(last, rows0, sem0, start=False)
        pool_one(last, rows0)
        gather(last + 1, rows1, sem1, start=False)
        pool_one(last + 1, rows1)

        pltpu.sync_copy(acc_v, out_hbm.at[pl.ds(base, bpw)])

    return pool(content, tw2d, b16)


def kernel(content, table, W, b):
    batch, seq = content.shape
    vocab, dim = table.shape
    label = W.shape[1]

    w16 = jnp.zeros((dim, _LANES), jnp.float32).at[:, :label].set(W)
    b16 = jnp.zeros((_LANES,), jnp.float32).at[:label].set(b)

    tw = _tc_table_times_w(table.T, w16)
    pooled = _sc_pool_tw(content.astype(jnp.int32),
                         tw.reshape(vocab, _LANES), b16,
                         batch=batch, seq=seq, vocab=vocab)
    return pooled[:, :label]


# speedup vs baseline: 2.6644x; 1.4143x over previous
import functools

import jax
import jax.numpy as jnp
from jax import lax
from jax.experimental import pallas as pl
from jax.experimental.pallas import tpu as pltpu
from jax.experimental.pallas import tpu_sc as plsc

_NUM_CORES = 2
_NUM_SUBCORES = 16
_LANES = 16


def _seq_chunks(seq_len):
    chunks = []
    off = 0
    while off < seq_len:
        size = min(128, seq_len - off)
        chunks.append((off, size))
        off += size
    return chunks


_CV = 16384
_SUB = _CV // 8


def _tc_table_times_w(tt, wbig):
    dim, vocab = tt.shape
    grid = (vocab + _CV - 1) // _CV

    def mm(tt_ref, w_ref, o_ref):
        acc = jax.lax.dot_general(
            tt_ref[:, pl.ds(0, _SUB)], w_ref[pl.ds(0, dim), :],
            dimension_numbers=(((0,), (0,)), ((), ())),
            preferred_element_type=jnp.float32)
        for j in range(1, 8):
            acc = acc + jax.lax.dot_general(
                tt_ref[:, pl.ds(j * _SUB, _SUB)],
                w_ref[pl.ds(j * dim, dim), :],
                dimension_numbers=(((0,), (0,)), ((), ())),
                preferred_element_type=jnp.float32)
        o_ref[...] = acc

    return pl.pallas_call(
        mm,
        grid=(grid,),
        in_specs=[
            pl.BlockSpec((dim, _CV), lambda i: (0, i)),
            pl.BlockSpec((8 * dim, 128), lambda i: (0, 0)),
        ],
        out_specs=pl.BlockSpec((_SUB, 128), lambda i: (i, 0)),
        out_shape=jax.ShapeDtypeStruct((grid * _SUB, 128), jnp.float32),
    )(tt, wbig)


@functools.partial(jax.jit, static_argnames=("batch", "seq"))
def _sc_pool_tw(content, tw2d, b16, *, batch, seq):
    lanes = _LANES
    n_workers = _NUM_CORES * _NUM_SUBCORES
    bpw = batch // n_workers
    assert bpw % 2 == 0
    assert (bpw * seq) % lanes == 0
    npair = bpw // 2
    chunks = _seq_chunks(seq)
    inv_s = jnp.float32(1.0 / seq)
    cv_bits = _CV.bit_length() - 1
    sub_bits = _SUB.bit_length() - 1

    mesh = plsc.VectorSubcoreMesh(
        core_axis_name="c", subcore_axis_name="s",
        num_cores=_NUM_CORES, num_subcores=_NUM_SUBCORES)

    @functools.partial(
        pl.kernel,
        mesh=mesh,
        out_type=jax.ShapeDtypeStruct((batch, lanes), jnp.float32),
        compiler_params=pltpu.CompilerParams(use_tc_tiling_on_sc=False),
        scratch_types=[
            pltpu.VMEM((bpw, seq), jnp.int32),
            pltpu.VMEM((seq, lanes), jnp.float32),
            pltpu.VMEM((seq, lanes), jnp.float32),
            pltpu.VMEM((bpw, lanes), jnp.float32),
            pltpu.VMEM((lanes,), jnp.float32),
            pltpu.SemaphoreType.DMA,
            pltpu.SemaphoreType.DMA,
        ],
    )
    def pool(content_hbm, tw2, b_hbm, out_hbm,
             idx_v, rows0, rows1, acc_v, b_v, sem0, sem1):
        wid = lax.axis_index("s") * _NUM_CORES + lax.axis_index("c")
        base = pl.multiple_of(wid * bpw, 8)

        pltpu.sync_copy(content_hbm.at[pl.ds(base, bpw)], idx_v)
        pltpu.sync_copy(b_hbm, b_v)
        bias = b_v[...]

        def sigma(v):
            return (((v >> cv_bits) << cv_bits)
                    + ((v & (_SUB - 1)) << 3)
                    + ((v >> sub_bits) & 7))

        rem = seq % lanes

        def remap(b, _):
            for k in range(seq // lanes):
                col = k * lanes
                idx_v[b, pl.ds(col, lanes)] = sigma(
                    idx_v[b, pl.ds(col, lanes)])
            if rem:
                x = idx_v[b, pl.ds(seq - lanes, lanes)]
                keep = lax.iota(jnp.int32, lanes) < (lanes - rem)
                idx_v[b, pl.ds(seq - lanes, lanes)] = jnp.where(
                    keep, x, sigma(x))
            return 0

        lax.fori_loop(0, bpw, remap, 0)

        def gather(b, buf, sem, start):
            for coff, csz in chunks:
                cpy = pltpu.make_async_copy(
                    tw2.at[idx_v.at[b, pl.ds(coff, csz)]],
                    buf.at[pl.ds(coff, csz)],
                    sem)
                if start:
                    cpy.start()
                else:
                    cpy.wait()

        def pool_one(b, buf):
            def body(i, acc):
                s0 = i * 8
                for j in range(8):
                    acc = acc + buf[s0 + j, :]
                return acc
            acc = lax.fori_loop(0, seq // 8, body,
                                jnp.zeros((lanes,), jnp.float32))
            for s in range(seq - seq % 8, seq):
                acc = acc + buf[s, :]
            acc_v[b, :] = acc * inv_s + bias

        gather(0, rows0, sem0, start=True)

        def pipeline(p, _):
            b0 = 2 * p
            gather(b0 + 1, rows1, sem1, start=True)
            gather(b0, rows0, sem0, start=False)
            pool_one(b0, rows0)
            gather(b0 + 2, rows0, sem0, start=True)
            gather(b0 + 1, rows1, sem1, start=False)
            pool_one(b0 + 1, rows1)
            return 0

        lax.fori_loop(0, npair - 1, pipeline, 0)

        last = bpw - 2
        gather(last + 1, rows1, sem1, start=True)
        gather(last, rows0, sem0, start=False)
        pool_one(last, rows0)
        gather(last + 1, rows1, sem1, start=False)
        pool_one(last + 1, rows1)

        pltpu.sync_copy(acc_v, out_hbm.at[pl.ds(base, bpw)])

    return pool(content, tw2d, b16)


def kernel(content, table, W, b):
    batch, seq = content.shape
    vocab, dim = table.shape
    label = W.shape[1]

    wbig = jnp.zeros((8 * dim, 128), jnp.float32)
    for j in range(8):
        wbig = wbig.at[j * dim:(j + 1) * dim,
                       j * _LANES:j * _LANES + label].set(W)
    b16 = jnp.zeros((_LANES,), jnp.float32).at[:label].set(b)

    tw = _tc_table_times_w(table.T, wbig)
    slots = tw.shape[0] * (128 // _LANES)
    pooled = _sc_pool_tw(content.astype(jnp.int32),
                         tw.reshape(slots, _LANES), b16,
                         batch=batch, seq=seq)
    return pooled[:, :label]


# speedup vs baseline: 3.1272x; 1.1737x over previous
import functools

import jax
import jax.numpy as jnp
from jax import lax
from jax.experimental import pallas as pl
from jax.experimental.pallas import tpu as pltpu
from jax.experimental.pallas import tpu_sc as plsc

_NUM_CORES = 2
_NUM_SUBCORES = 16
_LANES = 16


def _seq_chunks(seq_len):
    chunks = []
    off = 0
    while off < seq_len:
        size = min(128, seq_len - off)
        chunks.append((off, size))
        off += size
    return chunks


_CV = 16384
_SUB = _CV // 8


def _tc_table_times_w(tt, wbig):
    dim, vocab = tt.shape
    grid = (vocab + _CV - 1) // _CV

    def mm(tt_ref, w_ref, o_ref):
        lhs = jnp.concatenate(
            [tt_ref[:, pl.ds(j * _SUB, _SUB)] for j in range(8)],
            axis=0)
        o_ref[...] = jax.lax.dot_general(
            lhs, w_ref[...],
            dimension_numbers=(((0,), (0,)), ((), ())),
            preferred_element_type=jnp.float32)

    return pl.pallas_call(
        mm,
        grid=(grid,),
        in_specs=[
            pl.BlockSpec((dim, _CV), lambda i: (0, i)),
            pl.BlockSpec((8 * dim, 128), lambda i: (0, 0)),
        ],
        out_specs=pl.BlockSpec((_SUB, 128), lambda i: (i, 0)),
        out_shape=jax.ShapeDtypeStruct((grid * _SUB, 128), jnp.float32),
    )(tt, wbig)


@functools.partial(jax.jit, static_argnames=("batch", "seq"))
def _sc_pool_tw(content, tw2d, b16, *, batch, seq):
    lanes = _LANES
    n_workers = _NUM_CORES * _NUM_SUBCORES
    bpw = batch // n_workers
    assert bpw % 2 == 0
    assert (bpw * seq) % lanes == 0
    npair = bpw // 2
    chunks = _seq_chunks(seq)
    inv_s = jnp.float32(1.0 / seq)
    cv_bits = _CV.bit_length() - 1
    sub_bits = _SUB.bit_length() - 1

    mesh = plsc.VectorSubcoreMesh(
        core_axis_name="c", subcore_axis_name="s",
        num_cores=_NUM_CORES, num_subcores=_NUM_SUBCORES)

    @functools.partial(
        pl.kernel,
        mesh=mesh,
        out_type=jax.ShapeDtypeStruct((batch, lanes), jnp.float32),
        compiler_params=pltpu.CompilerParams(use_tc_tiling_on_sc=False),
        scratch_types=[
            pltpu.VMEM((bpw, seq), jnp.int32),
            pltpu.VMEM((seq, lanes), jnp.float32),
            pltpu.VMEM((seq, lanes), jnp.float32),
            pltpu.VMEM((bpw, lanes), jnp.float32),
            pltpu.VMEM((lanes,), jnp.float32),
            pltpu.SemaphoreType.DMA,
            pltpu.SemaphoreType.DMA,
        ],
    )
    def pool(content_hbm, tw2, b_hbm, out_hbm,
             idx_v, rows0, rows1, acc_v, b_v, sem0, sem1):
        wid = lax.axis_index("s") * _NUM_CORES + lax.axis_index("c")
        base = pl.multiple_of(wid * bpw, 8)

        pltpu.sync_copy(content_hbm.at[pl.ds(base, bpw)], idx_v)
        pltpu.sync_copy(b_hbm, b_v)
        bias = b_v[...]

        def sigma(v):
            return (((v >> cv_bits) << cv_bits)
                    + ((v & (_SUB - 1)) << 3)
                    + ((v >> sub_bits) & 7))

        rem = seq % lanes

        def remap(b, _):
            for k in range(seq // lanes):
                col = k * lanes
                idx_v[b, pl.ds(col, lanes)] = sigma(
                    idx_v[b, pl.ds(col, lanes)])
            if rem:
                x = idx_v[b, pl.ds(seq - lanes, lanes)]
                keep = lax.iota(jnp.int32, lanes) < (lanes - rem)
                idx_v[b, pl.ds(seq - lanes, lanes)] = jnp.where(
                    keep, x, sigma(x))
            return 0

        lax.fori_loop(0, bpw, remap, 0)

        def gather(b, buf, sem, start):
            for coff, csz in chunks:
                cpy = pltpu.make_async_copy(
                    tw2.at[idx_v.at[b, pl.ds(coff, csz)]],
                    buf.at[pl.ds(coff, csz)],
                    sem)
                if start:
                    cpy.start()
                else:
                    cpy.wait()

        def pool_one(b, buf):
            def body(i, acc):
                s0 = i * 8
                for j in range(8):
                    acc = acc + buf[s0 + j, :]
                return acc
            acc = lax.fori_loop(0, seq // 8, body,
                                jnp.zeros((lanes,), jnp.float32))
            for s in range(seq - seq % 8, seq):
                acc = acc + buf[s, :]
            acc_v[b, :] = acc * inv_s + bias

        gather(0, rows0, sem0, start=True)

        def pipeline(p, _):
            b0 = 2 * p
            gather(b0 + 1, rows1, sem1, start=True)
            gather(b0, rows0, sem0, start=False)
            pool_one(b0, rows0)
            gather(b0 + 2, rows0, sem0, start=True)
            gather(b0 + 1, rows1, sem1, start=False)
            pool_one(b0 + 1, rows1)
            return 0

        lax.fori_loop(0, npair - 1, pipeline, 0)

        last = bpw - 2
        gather(last + 1, rows1, sem1, start=True)
        gather(last, rows0, sem0, start=False)
        pool_one(last, rows0)
        gather(last + 1, rows1, sem1, start=False)
        pool_one(last + 1, rows1)

        pltpu.sync_copy(acc_v, out_hbm.at[pl.ds(base, bpw)])

    return pool(content, tw2d, b16)


def kernel(content, table, W, b):
    batch, seq = content.shape
    vocab, dim = table.shape
    label = W.shape[1]

    wbig = jnp.zeros((8 * dim, 128), jnp.float32)
    for j in range(8):
        wbig = wbig.at[j * dim:(j + 1) * dim,
                       j * _LANES:j * _LANES + label].set(W)
    b16 = jnp.zeros((_LANES,), jnp.float32).at[:label].set(b)

    tw = _tc_table_times_w(table.T, wbig)
    slots = tw.shape[0] * (128 // _LANES)
    pooled = _sc_pool_tw(content.astype(jnp.int32),
                         tw.reshape(slots, _LANES), b16,
                         batch=batch, seq=seq)
    return pooled[:, :label]


# speedup vs baseline: 3.2355x; 1.0346x over previous
import functools

import jax
import jax.numpy as jnp
from jax import lax
from jax.experimental import pallas as pl
from jax.experimental.pallas import tpu as pltpu
from jax.experimental.pallas import tpu_sc as plsc

_NUM_CORES = 2
_NUM_SUBCORES = 16
_LANES = 16


def _seq_chunks(seq_len):
    chunks = []
    off = 0
    while off < seq_len:
        size = min(128, seq_len - off)
        chunks.append((off, size))
        off += size
    return chunks


_CV = 16384
_SUB = _CV // 8


def _tc_table_times_w(tt, wbig):
    dim, vocab = tt.shape
    grid = (vocab + _CV - 1) // _CV

    def mm(tt_ref, w_ref, o_ref):
        lhs = jnp.concatenate(
            [tt_ref[:, pl.ds(j * _SUB, _SUB)] for j in range(8)],
            axis=0)
        o_ref[...] = jax.lax.dot_general(
            lhs, w_ref[...],
            dimension_numbers=(((0,), (0,)), ((), ())),
            preferred_element_type=jnp.float32)

    return pl.pallas_call(
        mm,
        grid=(grid,),
        in_specs=[
            pl.BlockSpec((dim, _CV), lambda i: (0, i)),
            pl.BlockSpec((8 * dim, 128), lambda i: (0, 0)),
        ],
        out_specs=pl.BlockSpec((_SUB, 128), lambda i: (i, 0)),
        out_shape=jax.ShapeDtypeStruct((grid * _SUB, 128), jnp.float32),
    )(tt, wbig)


@functools.partial(jax.jit, static_argnames=("batch", "seq"))
def _sc_pool_tw(content, tw2d, b16, *, batch, seq):
    lanes = _LANES
    n_workers = _NUM_CORES * _NUM_SUBCORES
    bpw = batch // n_workers
    assert bpw % 2 == 0
    assert (bpw * seq) % lanes == 0
    npair = bpw // 2
    chunks = _seq_chunks(seq)
    inv_s = jnp.float32(1.0 / seq)
    cv_bits = _CV.bit_length() - 1
    sub_bits = _SUB.bit_length() - 1

    mesh = plsc.VectorSubcoreMesh(
        core_axis_name="c", subcore_axis_name="s",
        num_cores=_NUM_CORES, num_subcores=_NUM_SUBCORES)

    @functools.partial(
        pl.kernel,
        mesh=mesh,
        out_type=jax.ShapeDtypeStruct((batch, lanes), jnp.float32),
        compiler_params=pltpu.CompilerParams(use_tc_tiling_on_sc=False),
        scratch_types=[
            pltpu.VMEM((bpw, seq), jnp.int32),
            pltpu.VMEM((seq, lanes), jnp.float32),
            pltpu.VMEM((seq, lanes), jnp.float32),
            pltpu.VMEM((bpw, lanes), jnp.float32),
            pltpu.VMEM((lanes,), jnp.float32),
            pltpu.SemaphoreType.DMA,
            pltpu.SemaphoreType.DMA,
        ],
    )
    def pool(content_hbm, tw2, b_hbm, out_hbm,
             idx_v, rows0, rows1, acc_v, b_v, sem0, sem1):
        wid = lax.axis_index("s") * _NUM_CORES + lax.axis_index("c")
        base = pl.multiple_of(wid * bpw, 8)

        pltpu.sync_copy(content_hbm.at[pl.ds(base, bpw)], idx_v)
        pltpu.sync_copy(b_hbm, b_v)
        bias = b_v[...]

        def sigma(v):
            return (((v >> cv_bits) << cv_bits)
                    + ((v & (_SUB - 1)) << 3)
                    + ((v >> sub_bits) & 7))

        rem = seq % lanes

        def remap(b, _):
            for k in range(seq // lanes):
                col = k * lanes
                idx_v[b, pl.ds(col, lanes)] = sigma(
                    idx_v[b, pl.ds(col, lanes)])
            if rem:
                x = idx_v[b, pl.ds(seq - lanes, lanes)]
                keep = lax.iota(jnp.int32, lanes) < (lanes - rem)
                idx_v[b, pl.ds(seq - lanes, lanes)] = jnp.where(
                    keep, x, sigma(x))
            return 0

        lax.fori_loop(0, bpw, remap, 0)

        def gather(b, buf, sem, start):
            for coff, csz in chunks:
                cpy = pltpu.make_async_copy(
                    tw2.at[idx_v.at[b, pl.ds(coff, csz)]],
                    buf.at[pl.ds(coff, csz)],
                    sem)
                if start:
                    cpy.start()
                else:
                    cpy.wait()

        def pool_one(b, buf):
            def body(i, accs):
                s0 = i * 8
                a0, a1, a2, a3 = accs
                for j in range(0, 8, 4):
                    a0 = a0 + buf[s0 + j, :]
                    a1 = a1 + buf[s0 + j + 1, :]
                    a2 = a2 + buf[s0 + j + 2, :]
                    a3 = a3 + buf[s0 + j + 3, :]
                return a0, a1, a2, a3
            zero = jnp.zeros((lanes,), jnp.float32)
            a0, a1, a2, a3 = lax.fori_loop(0, seq // 8, body,
                                           (zero, zero, zero, zero))
            acc = (a0 + a1) + (a2 + a3)
            for s in range(seq - seq % 8, seq):
                acc = acc + buf[s, :]
            acc_v[b, :] = acc * inv_s + bias

        gather(0, rows0, sem0, start=True)

        def pipeline(p, _):
            b0 = 2 * p
            gather(b0 + 1, rows1, sem1, start=True)
            gather(b0, rows0, sem0, start=False)
            pool_one(b0, rows0)
            gather(b0 + 2, rows0, sem0, start=True)
            gather(b0 + 1, rows1, sem1, start=False)
            pool_one(b0 + 1, rows1)
            return 0

        lax.fori_loop(0, npair - 1, pipeline, 0)

        last = bpw - 2
        gather(last + 1, rows1, sem1, start=True)
        gather(last, rows0, sem0, start=False)
        pool_one(last, rows0)
        gather(last + 1, rows1, sem1, start=False)
        pool_one(last + 1, rows1)

        pltpu.sync_copy(acc_v, out_hbm.at[pl.ds(base, bpw)])

    return pool(content, tw2d, b16)


def kernel(content, table, W, b):
    batch, seq = content.shape
    vocab, dim = table.shape
    label = W.shape[1]

    wbig = jnp.zeros((8 * dim, 128), jnp.float32)
    for j in range(8):
        wbig = wbig.at[j * dim:(j + 1) * dim,
                       j * _LANES:j * _LANES + label].set(W)
    b16 = jnp.zeros((_LANES,), jnp.float32).at[:label].set(b)

    tw = _tc_table_times_w(table.T, wbig)
    slots = tw.shape[0] * (128 // _LANES)
    pooled = _sc_pool_tw(content.astype(jnp.int32),
                         tw.reshape(slots, _LANES), b16,
                         batch=batch, seq=seq)
    return pooled[:, :label]


# speedup vs baseline: 3.2412x; 1.0018x over previous
import functools

import jax
import jax.numpy as jnp
from jax import lax
from jax.experimental import pallas as pl
from jax.experimental.pallas import tpu as pltpu
from jax.experimental.pallas import tpu_sc as plsc

_NUM_CORES = 2
_NUM_SUBCORES = 16
_LANES = 16


def _seq_chunks(seq_len):
    chunks = []
    off = 0
    while off < seq_len:
        size = min(128, seq_len - off)
        chunks.append((off, size))
        off += size
    return chunks


_CV = 16384
_SUB = _CV // 8


def _tc_table_times_w(tt, wbig):
    dim, vocab = tt.shape
    grid = (vocab + _CV - 1) // _CV

    def mm(tt_ref, w_ref, o_ref):
        lhs = jnp.concatenate(
            [tt_ref[:, pl.ds(j * _SUB, _SUB)] for j in range(8)],
            axis=0)
        o_ref[...] = jax.lax.dot_general(
            lhs, w_ref[...],
            dimension_numbers=(((0,), (0,)), ((), ())),
            preferred_element_type=jnp.float32)

    return pl.pallas_call(
        mm,
        grid=(grid,),
        in_specs=[
            pl.BlockSpec((dim, _CV), lambda i: (0, i)),
            pl.BlockSpec((8 * dim, 128), lambda i: (0, 0)),
        ],
        out_specs=pl.BlockSpec((_SUB, 128), lambda i: (i, 0)),
        out_shape=jax.ShapeDtypeStruct((grid * _SUB, 128), jnp.float32),
    )(tt, wbig)


@functools.partial(jax.jit, static_argnames=("batch", "seq"))
def _sc_pool_tw(content, tw2d, b16, *, batch, seq):
    lanes = _LANES
    n_workers = _NUM_CORES * _NUM_SUBCORES
    bpw = batch // n_workers
    assert bpw % 2 == 0
    assert (bpw * seq) % lanes == 0
    npair = bpw // 2
    chunks = _seq_chunks(seq)
    inv_s = jnp.float32(1.0 / seq)
    cv_bits = _CV.bit_length() - 1
    sub_bits = _SUB.bit_length() - 1

    mesh = plsc.VectorSubcoreMesh(
        core_axis_name="c", subcore_axis_name="s",
        num_cores=_NUM_CORES, num_subcores=_NUM_SUBCORES)

    @functools.partial(
        pl.kernel,
        mesh=mesh,
        out_type=jax.ShapeDtypeStruct((batch, lanes), jnp.float32),
        compiler_params=pltpu.CompilerParams(use_tc_tiling_on_sc=False),
        scratch_types=[
            pltpu.VMEM((bpw, seq), jnp.int32),
            pltpu.VMEM((seq, lanes), jnp.float32),
            pltpu.VMEM((seq, lanes), jnp.float32),
            pltpu.VMEM((bpw, lanes), jnp.float32),
            pltpu.VMEM((lanes,), jnp.float32),
            pltpu.SemaphoreType.DMA,
            pltpu.SemaphoreType.DMA,
        ],
    )
    def pool(content_hbm, tw2, b_hbm, out_hbm,
             idx_v, rows0, rows1, acc_v, b_v, sem0, sem1):
        wid = lax.axis_index("s") * _NUM_CORES + lax.axis_index("c")
        base = pl.multiple_of(wid * bpw, 8)

        pltpu.sync_copy(content_hbm.at[pl.ds(base, bpw)], idx_v)
        pltpu.sync_copy(b_hbm, b_v)
        bias = b_v[...]

        def sigma(v):
            return (((v >> cv_bits) << cv_bits)
                    + ((v & (_SUB - 1)) << 3)
                    + ((v >> sub_bits) & 7))

        rem = seq % lanes

        def remap(b, _):
            for k in range(seq // lanes):
                col = k * lanes
                idx_v[b, pl.ds(col, lanes)] = sigma(
                    idx_v[b, pl.ds(col, lanes)])
            if rem:
                x = idx_v[b, pl.ds(seq - lanes, lanes)]
                keep = lax.iota(jnp.int32, lanes) < (lanes - rem)
                idx_v[b, pl.ds(seq - lanes, lanes)] = jnp.where(
                    keep, x, sigma(x))
            return 0

        lax.fori_loop(0, bpw, remap, 0)

        def gather(b, buf, sem, start):
            for coff, csz in chunks:
                cpy = pltpu.make_async_copy(
                    tw2.at[idx_v.at[b, pl.ds(coff, csz)]],
                    buf.at[pl.ds(coff, csz)],
                    sem)
                if start:
                    cpy.start()
                else:
                    cpy.wait()

        def pool_one(b, buf):
            def body(i, accs):
                s0 = i * 8
                a0, a1, a2, a3 = accs
                for j in range(0, 8, 4):
                    a0 = a0 + buf[s0 + j, :]
                    a1 = a1 + buf[s0 + j + 1, :]
                    a2 = a2 + buf[s0 + j + 2, :]
                    a3 = a3 + buf[s0 + j + 3, :]
                return a0, a1, a2, a3
            zero = jnp.zeros((lanes,), jnp.float32)
            a0, a1, a2, a3 = plsc.parallel_loop(
                0, seq // 8, unroll=4,
                carry=(zero, zero, zero, zero))(body)
            acc = (a0 + a1) + (a2 + a3)
            for s in range(seq - seq % 8, seq):
                acc = acc + buf[s, :]
            acc_v[b, :] = acc * inv_s + bias

        gather(0, rows0, sem0, start=True)

        def pipeline(p, _):
            b0 = 2 * p
            gather(b0 + 1, rows1, sem1, start=True)
            gather(b0, rows0, sem0, start=False)
            pool_one(b0, rows0)
            gather(b0 + 2, rows0, sem0, start=True)
            gather(b0 + 1, rows1, sem1, start=False)
            pool_one(b0 + 1, rows1)
            return 0

        lax.fori_loop(0, npair - 1, pipeline, 0)

        last = bpw - 2
        gather(last + 1, rows1, sem1, start=True)
        gather(last, rows0, sem0, start=False)
        pool_one(last, rows0)
        gather(last + 1, rows1, sem1, start=False)
        pool_one(last + 1, rows1)

        pltpu.sync_copy(acc_v, out_hbm.at[pl.ds(base, bpw)])

    return pool(content, tw2d, b16)


def kernel(content, table, W, b):
    batch, seq = content.shape
    vocab, dim = table.shape
    label = W.shape[1]

    wbig = jnp.zeros((8 * dim, 128), jnp.float32)
    for j in range(8):
        wbig = wbig.at[j * dim:(j + 1) * dim,
                       j * _LANES:j * _LANES + label].set(W)
    b16 = jnp.zeros((_LANES,), jnp.float32).at[:label].set(b)

    tw = _tc_table_times_w(table.T, wbig)
    slots = tw.shape[0] * (128 // _LANES)
    pooled = _sc_pool_tw(content.astype(jnp.int32),
                         tw.reshape(slots, _LANES), b16,
                         batch=batch, seq=seq)
    return pooled[:, :label]


# speedup vs baseline: 3.6974x; 1.1407x over previous
import functools

import jax
import jax.numpy as jnp
from jax import lax
from jax.experimental import pallas as pl
from jax.experimental.pallas import tpu as pltpu
from jax.experimental.pallas import tpu_sc as plsc

_NUM_CORES = 2
_NUM_SUBCORES = 16
_LANES = 16


def _seq_chunks(seq_len):
    chunks = []
    off = 0
    while off < seq_len:
        size = min(128, seq_len - off)
        chunks.append((off, size))
        off += size
    return chunks


_CV = 16384
_SUB = _CV // 8


def _tc_table_times_w(tt, wbig):
    dim, vocab = tt.shape
    grid = (vocab + _CV - 1) // _CV

    def mm(tt_ref, w_ref, o_ref):
        lhs = jnp.concatenate(
            [tt_ref[:, pl.ds(j * _SUB, _SUB)] for j in range(8)],
            axis=0)
        o_ref[...] = jax.lax.dot_general(
            lhs, w_ref[...],
            dimension_numbers=(((0,), (0,)), ((), ())),
            preferred_element_type=jnp.float32)

    return pl.pallas_call(
        mm,
        grid=(grid,),
        in_specs=[
            pl.BlockSpec((dim, _CV), lambda i: (0, i)),
            pl.BlockSpec((8 * dim, 128), lambda i: (0, 0)),
        ],
        out_specs=pl.BlockSpec((_SUB, 128), lambda i: (i, 0)),
        out_shape=jax.ShapeDtypeStruct((grid * _SUB, 128), jnp.float32),
    )(tt, wbig)


@functools.partial(jax.jit, static_argnames=("batch", "seq"))
def _sc_pool_tw(content, tw2d, b16, *, batch, seq):
    lanes = _LANES
    n_workers = _NUM_CORES * _NUM_SUBCORES
    bpw = batch // n_workers
    assert bpw % 2 == 0
    assert (bpw * seq) % lanes == 0
    npair = bpw // 2
    chunks = _seq_chunks(seq)
    inv_s = jnp.float32(1.0 / seq)
    cv_bits = _CV.bit_length() - 1
    sub_bits = _SUB.bit_length() - 1

    mesh = plsc.VectorSubcoreMesh(
        core_axis_name="c", subcore_axis_name="s",
        num_cores=_NUM_CORES, num_subcores=_NUM_SUBCORES)

    @functools.partial(
        pl.kernel,
        mesh=mesh,
        out_type=jax.ShapeDtypeStruct((batch, lanes), jnp.float32),
        compiler_params=pltpu.CompilerParams(use_tc_tiling_on_sc=False),
        scratch_types=[
            pltpu.VMEM((bpw, seq), jnp.int32),
            pltpu.VMEM((seq, lanes), jnp.float32),
            pltpu.VMEM((seq, lanes), jnp.float32),
            pltpu.VMEM((seq, lanes), jnp.float32),
            pltpu.VMEM((seq, lanes), jnp.float32),
            pltpu.VMEM((bpw, lanes), jnp.float32),
            pltpu.VMEM((lanes,), jnp.float32),
            pltpu.SemaphoreType.DMA,
            pltpu.SemaphoreType.DMA,
            pltpu.SemaphoreType.DMA,
            pltpu.SemaphoreType.DMA,
        ],
    )
    def pool(content_hbm, tw2, b_hbm, out_hbm,
             idx_v, rows0, rows1, rows2, rows3, acc_v, b_v,
             sem0, sem1, sem2, sem3):
        wid = lax.axis_index("s") * _NUM_CORES + lax.axis_index("c")
        base = pl.multiple_of(wid * bpw, 8)

        pltpu.sync_copy(content_hbm.at[pl.ds(base, bpw)], idx_v)
        pltpu.sync_copy(b_hbm, b_v)
        bias = b_v[...]

        def sigma(v):
            return (((v >> cv_bits) << cv_bits)
                    + ((v & (_SUB - 1)) << 3)
                    + ((v >> sub_bits) & 7))

        rem = seq % lanes

        def remap(b, _):
            for k in range(seq // lanes):
                col = k * lanes
                idx_v[b, pl.ds(col, lanes)] = sigma(
                    idx_v[b, pl.ds(col, lanes)])
            if rem:
                x = idx_v[b, pl.ds(seq - lanes, lanes)]
                keep = lax.iota(jnp.int32, lanes) < (lanes - rem)
                idx_v[b, pl.ds(seq - lanes, lanes)] = jnp.where(
                    keep, x, sigma(x))
            return 0

        lax.fori_loop(0, bpw, remap, 0)

        def gather(b, buf, sem, start):
            for coff, csz in chunks:
                cpy = pltpu.make_async_copy(
                    tw2.at[idx_v.at[b, pl.ds(coff, csz)]],
                    buf.at[pl.ds(coff, csz)],
                    sem)
                if start:
                    cpy.start()
                else:
                    cpy.wait()

        def pool_one(b, buf):
            def body(i, accs):
                s0 = i * 8
                a0, a1, a2, a3 = accs
                for j in range(0, 8, 4):
                    a0 = a0 + buf[s0 + j, :]
                    a1 = a1 + buf[s0 + j + 1, :]
                    a2 = a2 + buf[s0 + j + 2, :]
                    a3 = a3 + buf[s0 + j + 3, :]
                return a0, a1, a2, a3
            zero = jnp.zeros((lanes,), jnp.float32)
            a0, a1, a2, a3 = plsc.parallel_loop(
                0, seq // 8, unroll=4,
                carry=(zero, zero, zero, zero))(body)
            acc = (a0 + a1) + (a2 + a3)
            for s in range(seq - seq % 8, seq):
                acc = acc + buf[s, :]
            acc_v[b, :] = acc * inv_s + bias

        bufs = ((rows0, sem0), (rows1, sem1), (rows2, sem2), (rows3, sem3))
        for q in range(3):
            gather(q, bufs[q][0], bufs[q][1], start=True)

        def pipeline(g, _):
            b = 4 * g
            for q in range(4):
                nbuf, nsem = bufs[(q + 3) % 4]
                gather(b + q + 3, nbuf, nsem, start=True)
                buf, sem = bufs[q]
                gather(b + q, buf, sem, start=False)
                pool_one(b + q, buf)
            return 0

        lax.fori_loop(0, bpw // 4 - 1, pipeline, 0)

        b = bpw - 4
        gather(b + 3, rows3, sem3, start=True)
        for q in range(4):
            buf, sem = bufs[q]
            gather(b + q, buf, sem, start=False)
            pool_one(b + q, buf)

        pltpu.sync_copy(acc_v, out_hbm.at[pl.ds(base, bpw)])

    return pool(content, tw2d, b16)


def kernel(content, table, W, b):
    batch, seq = content.shape
    vocab, dim = table.shape
    label = W.shape[1]

    wbig = jnp.zeros((8 * dim, 128), jnp.float32)
    for j in range(8):
        wbig = wbig.at[j * dim:(j + 1) * dim,
                       j * _LANES:j * _LANES + label].set(W)
    b16 = jnp.zeros((_LANES,), jnp.float32).at[:label].set(b)

    tw = _tc_table_times_w(table.T, wbig)
    slots = tw.shape[0] * (128 // _LANES)
    pooled = _sc_pool_tw(content.astype(jnp.int32),
                         tw.reshape(slots, _LANES), b16,
                         batch=batch, seq=seq)
    return pooled[:, :label]


# speedup vs baseline: 3.7865x; 1.0241x over previous
import functools

import jax
import jax.numpy as jnp
from jax import lax
from jax.experimental import pallas as pl
from jax.experimental.pallas import tpu as pltpu
from jax.experimental.pallas import tpu_sc as plsc

_NUM_CORES = 2
_NUM_SUBCORES = 16
_LANES = 16
_NBUF = 8


def _seq_chunks(seq_len):
    chunks = []
    off = 0
    while off < seq_len:
        size = min(128, seq_len - off)
        chunks.append((off, size))
        off += size
    return chunks


_CV = 16384
_SUB = _CV // 8


def _tc_table_times_w(tt, wbig):
    dim, vocab = tt.shape
    grid = (vocab + _CV - 1) // _CV

    def mm(tt_ref, w_ref, o_ref):
        lhs = jnp.concatenate(
            [tt_ref[:, pl.ds(j * _SUB, _SUB)] for j in range(8)],
            axis=0)
        o_ref[...] = jax.lax.dot_general(
            lhs, w_ref[...],
            dimension_numbers=(((0,), (0,)), ((), ())),
            preferred_element_type=jnp.float32)

    return pl.pallas_call(
        mm,
        grid=(grid,),
        in_specs=[
            pl.BlockSpec((dim, _CV), lambda i: (0, i)),
            pl.BlockSpec((8 * dim, 128), lambda i: (0, 0)),
        ],
        out_specs=pl.BlockSpec((_SUB, 128), lambda i: (i, 0)),
        out_shape=jax.ShapeDtypeStruct((grid * _SUB, 128), jnp.float32),
    )(tt, wbig)


@functools.partial(jax.jit, static_argnames=("batch", "seq"))
def _sc_pool_tw(content, tw2d, b16, *, batch, seq):
    lanes = _LANES
    n_workers = _NUM_CORES * _NUM_SUBCORES
    bpw = batch // n_workers
    assert bpw % 2 == 0
    assert (bpw * seq) % lanes == 0
    npair = bpw // 2
    chunks = _seq_chunks(seq)
    inv_s = jnp.float32(1.0 / seq)
    cv_bits = _CV.bit_length() - 1
    sub_bits = _SUB.bit_length() - 1

    mesh = plsc.VectorSubcoreMesh(
        core_axis_name="c", subcore_axis_name="s",
        num_cores=_NUM_CORES, num_subcores=_NUM_SUBCORES)

    @functools.partial(
        pl.kernel,
        mesh=mesh,
        out_type=jax.ShapeDtypeStruct((batch, lanes), jnp.float32),
        compiler_params=pltpu.CompilerParams(use_tc_tiling_on_sc=False),
        scratch_types=[
            pltpu.VMEM((bpw, seq), jnp.int32),
        ] + [
            pltpu.VMEM((seq, lanes), jnp.float32)
            for _ in range(_NBUF)
        ] + [
            pltpu.VMEM((bpw, lanes), jnp.float32),
            pltpu.VMEM((lanes,), jnp.float32),
        ] + [pltpu.SemaphoreType.DMA for _ in range(_NBUF)],
    )
    def pool(content_hbm, tw2, b_hbm, out_hbm, idx_v, *rest):
        rows = rest[:_NBUF]
        acc_v, b_v = rest[_NBUF], rest[_NBUF + 1]
        sems = rest[_NBUF + 2:]
        wid = lax.axis_index("s") * _NUM_CORES + lax.axis_index("c")
        base = pl.multiple_of(wid * bpw, 8)

        pltpu.sync_copy(content_hbm.at[pl.ds(base, bpw)], idx_v)
        pltpu.sync_copy(b_hbm, b_v)
        bias = b_v[...]

        def sigma(v):
            return (((v >> cv_bits) << cv_bits)
                    + ((v & (_SUB - 1)) << 3)
                    + ((v >> sub_bits) & 7))

        rem = seq % lanes

        def remap(b, _):
            for k in range(seq // lanes):
                col = k * lanes
                idx_v[b, pl.ds(col, lanes)] = sigma(
                    idx_v[b, pl.ds(col, lanes)])
            if rem:
                x = idx_v[b, pl.ds(seq - lanes, lanes)]
                keep = lax.iota(jnp.int32, lanes) < (lanes - rem)
                idx_v[b, pl.ds(seq - lanes, lanes)] = jnp.where(
                    keep, x, sigma(x))
            return 0

        lax.fori_loop(0, bpw, remap, 0)

        def gather(b, buf, sem, start):
            for coff, csz in chunks:
                cpy = pltpu.make_async_copy(
                    tw2.at[idx_v.at[b, pl.ds(coff, csz)]],
                    buf.at[pl.ds(coff, csz)],
                    sem)
                if start:
                    cpy.start()
                else:
                    cpy.wait()

        def pool_one(b, buf):
            def body(i, accs):
                s0 = i * 8
                a0, a1, a2, a3 = accs
                for j in range(0, 8, 4):
                    a0 = a0 + buf[s0 + j, :]
                    a1 = a1 + buf[s0 + j + 1, :]
                    a2 = a2 + buf[s0 + j + 2, :]
                    a3 = a3 + buf[s0 + j + 3, :]
                return a0, a1, a2, a3
            zero = jnp.zeros((lanes,), jnp.float32)
            a0, a1, a2, a3 = plsc.parallel_loop(
                0, seq // 8, unroll=4,
                carry=(zero, zero, zero, zero))(body)
            acc = (a0 + a1) + (a2 + a3)
            for s in range(seq - seq % 8, seq):
                acc = acc + buf[s, :]
            acc_v[b, :] = acc * inv_s + bias

        nb = _NBUF
        bufs = tuple(zip(rows, sems))
        for q in range(nb - 1):
            gather(q, bufs[q][0], bufs[q][1], start=True)

        def pipeline(g, _):
            b = nb * g
            for q in range(nb):
                nbuf, nsem = bufs[(q + nb - 1) % nb]
                gather(b + q + nb - 1, nbuf, nsem, start=True)
                buf, sem = bufs[q]
                gather(b + q, buf, sem, start=False)
                pool_one(b + q, buf)
            return 0

        lax.fori_loop(0, bpw // nb - 1, pipeline, 0)

        b = bpw - nb
        gather(b + nb - 1, bufs[nb - 1][0], bufs[nb - 1][1], start=True)
        for q in range(nb):
            buf, sem = bufs[q]
            gather(b + q, buf, sem, start=False)
            pool_one(b + q, buf)

        pltpu.sync_copy(acc_v, out_hbm.at[pl.ds(base, bpw)])

    return pool(content, tw2d, b16)


def kernel(content, table, W, b):
    batch, seq = content.shape
    vocab, dim = table.shape
    label = W.shape[1]

    wbig = jnp.zeros((8 * dim, 128), jnp.float32)
    for j in range(8):
        wbig = wbig.at[j * dim:(j + 1) * dim,
                       j * _LANES:j * _LANES + label].set(W)
    b16 = jnp.zeros((_LANES,), jnp.float32).at[:label].set(b)

    tw = _tc_table_times_w(table.T, wbig)
    slots = tw.shape[0] * (128 // _LANES)
    pooled = _sc_pool_tw(content.astype(jnp.int32),
                         tw.reshape(slots, _LANES), b16,
                         batch=batch, seq=seq)
    return pooled[:, :label]


# speedup vs baseline: 4.1488x; 1.0957x over previous
import functools

import jax
import jax.numpy as jnp
from jax import lax
from jax.experimental import pallas as pl
from jax.experimental.pallas import tpu as pltpu
from jax.experimental.pallas import tpu_sc as plsc

_NUM_CORES = 2
_NUM_SUBCORES = 16
_LANES = 16
_NBUF = 8


def _seq_chunks(seq_len):
    chunks = []
    off = 0
    while off < seq_len:
        size = min(128, seq_len - off)
        chunks.append((off, size))
        off += size
    return chunks


_CV = 32768
_SUB = _CV // 8


def _tc_table_times_w(tt, wbig):
    dim, vocab = tt.shape
    grid = (vocab + _CV - 1) // _CV

    def mm(tt_ref, w_ref, o_ref):
        lhs = jnp.concatenate(
            [tt_ref[:, pl.ds(j * _SUB, _SUB)] for j in range(8)],
            axis=0)
        o_ref[...] = jax.lax.dot_general(
            lhs, w_ref[...],
            dimension_numbers=(((0,), (0,)), ((), ())),
            preferred_element_type=jnp.float32)

    return pl.pallas_call(
        mm,
        grid=(grid,),
        in_specs=[
            pl.BlockSpec((dim, _CV), lambda i: (0, i)),
            pl.BlockSpec((8 * dim, 128), lambda i: (0, 0)),
        ],
        out_specs=pl.BlockSpec((_SUB, 128), lambda i: (i, 0)),
        out_shape=jax.ShapeDtypeStruct((grid * _SUB, 128), jnp.float32),
    )(tt, wbig)


@functools.partial(jax.jit, static_argnames=("batch", "seq"))
def _sc_pool_tw(content, tw2d, b16, *, batch, seq):
    lanes = _LANES
    n_workers = _NUM_CORES * _NUM_SUBCORES
    bpw = batch // n_workers
    assert bpw % 2 == 0
    assert (bpw * seq) % lanes == 0
    npair = bpw // 2
    chunks = _seq_chunks(seq)
    inv_s = jnp.float32(1.0 / seq)
    cv_bits = _CV.bit_length() - 1
    sub_bits = _SUB.bit_length() - 1

    mesh = plsc.VectorSubcoreMesh(
        core_axis_name="c", subcore_axis_name="s",
        num_cores=_NUM_CORES, num_subcores=_NUM_SUBCORES)

    @functools.partial(
        pl.kernel,
        mesh=mesh,
        out_type=jax.ShapeDtypeStruct((batch, lanes), jnp.float32),
        compiler_params=pltpu.CompilerParams(use_tc_tiling_on_sc=False),
        scratch_types=[
            pltpu.VMEM((bpw, seq), jnp.int32),
        ] + [
            pltpu.VMEM((seq, lanes), jnp.float32)
            for _ in range(_NBUF)
        ] + [
            pltpu.VMEM((bpw, lanes), jnp.float32),
            pltpu.VMEM((lanes,), jnp.float32),
        ] + [pltpu.SemaphoreType.DMA for _ in range(_NBUF)],
    )
    def pool(content_hbm, tw2, b_hbm, out_hbm, idx_v, *rest):
        rows = rest[:_NBUF]
        acc_v, b_v = rest[_NBUF], rest[_NBUF + 1]
        sems = rest[_NBUF + 2:]
        wid = lax.axis_index("s") * _NUM_CORES + lax.axis_index("c")
        base = pl.multiple_of(wid * bpw, 8)

        pltpu.sync_copy(content_hbm.at[pl.ds(base, bpw)], idx_v)
        pltpu.sync_copy(b_hbm, b_v)
        bias = b_v[...]

        def sigma(v):
            return (((v >> cv_bits) << cv_bits)
                    + ((v & (_SUB - 1)) << 3)
                    + ((v >> sub_bits) & 7))

        rem = seq % lanes

        def remap(b, _):
            for k in range(seq // lanes):
                col = k * lanes
                idx_v[b, pl.ds(col, lanes)] = sigma(
                    idx_v[b, pl.ds(col, lanes)])
            if rem:
                x = idx_v[b, pl.ds(seq - lanes, lanes)]
                keep = lax.iota(jnp.int32, lanes) < (lanes - rem)
                idx_v[b, pl.ds(seq - lanes, lanes)] = jnp.where(
                    keep, x, sigma(x))
            return 0

        lax.fori_loop(0, bpw, remap, 0)

        def gather(b, buf, sem, start):
            for coff, csz in chunks:
                cpy = pltpu.make_async_copy(
                    tw2.at[idx_v.at[b, pl.ds(coff, csz)]],
                    buf.at[pl.ds(coff, csz)],
                    sem)
                if start:
                    cpy.start()
                else:
                    cpy.wait()

        def pool_one(b, buf):
            def body(i, accs):
                s0 = i * 8
                a0, a1, a2, a3 = accs
                for j in range(0, 8, 4):
                    a0 = a0 + buf[s0 + j, :]
                    a1 = a1 + buf[s0 + j + 1, :]
                    a2 = a2 + buf[s0 + j + 2, :]
                    a3 = a3 + buf[s0 + j + 3, :]
                return a0, a1, a2, a3
            zero = jnp.zeros((lanes,), jnp.float32)
            a0, a1, a2, a3 = plsc.parallel_loop(
                0, seq // 8, unroll=4,
                carry=(zero, zero, zero, zero))(body)
            acc = (a0 + a1) + (a2 + a3)
            for s in range(seq - seq % 8, seq):
                acc = acc + buf[s, :]
            acc_v[b, :] = acc * inv_s + bias

        nb = _NBUF
        bufs = tuple(zip(rows, sems))
        for q in range(nb - 1):
            gather(q, bufs[q][0], bufs[q][1], start=True)

        def pipeline(g, _):
            b = nb * g
            for q in range(nb):
                nbuf, nsem = bufs[(q + nb - 1) % nb]
                gather(b + q + nb - 1, nbuf, nsem, start=True)
                buf, sem = bufs[q]
                gather(b + q, buf, sem, start=False)
                pool_one(b + q, buf)
            return 0

        lax.fori_loop(0, bpw // nb - 1, pipeline, 0)

        b = bpw - nb
        gather(b + nb - 1, bufs[nb - 1][0], bufs[nb - 1][1], start=True)
        for q in range(nb):
            buf, sem = bufs[q]
            gather(b + q, buf, sem, start=False)
            pool_one(b + q, buf)

        pltpu.sync_copy(acc_v, out_hbm.at[pl.ds(base, bpw)])

    return pool(content, tw2d, b16)


def kernel(content, table, W, b):
    batch, seq = content.shape
    vocab, dim = table.shape
    label = W.shape[1]

    wbig = jnp.zeros((8 * dim, 128), jnp.float32)
    for j in range(8):
        wbig = wbig.at[j * dim:(j + 1) * dim,
                       j * _LANES:j * _LANES + label].set(W)
    b16 = jnp.zeros((_LANES,), jnp.float32).at[:label].set(b)

    tw = _tc_table_times_w(table.T, wbig)
    slots = tw.shape[0] * (128 // _LANES)
    pooled = _sc_pool_tw(content.astype(jnp.int32),
                         tw.reshape(slots, _LANES), b16,
                         batch=batch, seq=seq)
    return pooled[:, :label]
